# Initial kernel scaffold; baseline (speedup 1.0000x reference)
import jax
import jax.numpy as jnp
from jax.experimental import pallas as pl


def kernel(h, edge_index, W, b):
    raise NotImplementedError("write your pallas kernel here")


# speedup vs baseline: 2.9076x; 2.9076x over previous
import functools

import jax
import jax.numpy as jnp
from jax import lax
from jax.experimental import pallas as pl
from jax.experimental.pallas import tpu as pltpu
from jax.experimental.pallas import tpu_sc as plsc

N_NODES = 10000
N_EDGES = 320000
D_FEAT = 128
D_OUT = 128

NC = 2
NS = 16
NW = NC * NS
CHUNK = 64
CPT = 160
GRP = 8
NGRP = CPT // GRP
E_PAD = NW * CPT * CHUNK
N_PAD = 10112
ROWS_PER_TILE = N_PAD // NS


def _sc_feat_body(h_hbm, idx_hbm, zeros_hbm, sums_out, si_di, rb0, acc_sh,
                  sem):
    c = lax.axis_index("c")
    s = lax.axis_index("s")
    wid = c * NS + s

    pltpu.sync_copy(zeros_hbm, rb0)
    r0 = s * ROWS_PER_TILE
    nblk = (ROWS_PER_TILE + CHUNK - 1) // CHUNK
    for k in range(nblk):
        n = min(CHUNK, ROWS_PER_TILE - CHUNK * k)
        pltpu.sync_copy(rb0.at[pl.ds(0, n)],
                        acc_sh.at[pl.ds(r0 + CHUNK * k, n)])

    plsc.subcore_barrier()

    def _group(g, _):
        pltpu.sync_copy(idx_hbm.at[wid, g], si_di)
        for j in range(GRP):
            pltpu.async_copy(h_hbm.at[si_di.at[j]], rb0, sem).wait()
            pltpu.sync_copy(rb0, acc_sh.at[si_di.at[GRP + j]], add=True)
        return 0

    lax.fori_loop(0, NGRP, _group, 0)

    plsc.subcore_barrier()

    pltpu.sync_copy(acc_sh.at[pl.ds(r0, ROWS_PER_TILE)],
                    sums_out.at[c, pl.ds(r0, ROWS_PER_TILE)])


_sc_feat = functools.partial(
    pl.kernel,
    out_type=jax.ShapeDtypeStruct((NC, N_PAD, D_FEAT), jnp.float32),
    mesh=plsc.VectorSubcoreMesh(core_axis_name="c", subcore_axis_name="s"),
    scratch_types=(
        pltpu.VMEM((2 * GRP, CHUNK), jnp.int32),
        pltpu.VMEM((CHUNK, D_FEAT), jnp.float32),
        pltpu.VMEM_SHARED((N_PAD, D_FEAT), jnp.float32),
        pltpu.SemaphoreType.DMA,
    ),
)(_sc_feat_body)


def _sc_deg_body(idx_hbm, zeros_hbm, ones_hbm, deg_out, si_di, ob, deg_sh):
    c = lax.axis_index("c")
    s = lax.axis_index("s")
    wid = c * NS + s

    pltpu.sync_copy(zeros_hbm, ob)
    r0 = s * ROWS_PER_TILE
    nblk = (ROWS_PER_TILE + CHUNK - 1) // CHUNK
    for k in range(nblk):
        n = min(CHUNK, ROWS_PER_TILE - CHUNK * k)
        pltpu.sync_copy(ob.at[pl.ds(0, n)],
                        deg_sh.at[pl.ds(r0 + CHUNK * k, n)])
    pltpu.sync_copy(ones_hbm, ob)

    plsc.subcore_barrier()

    def _group(g, _):
        pltpu.sync_copy(idx_hbm.at[wid, g], si_di)
        for j in range(GRP):
            pltpu.sync_copy(ob, deg_sh.at[si_di.at[GRP + j]], add=True)
        return 0

    lax.fori_loop(0, NGRP, _group, 0)

    plsc.subcore_barrier()

    pltpu.sync_copy(deg_sh.at[pl.ds(r0, ROWS_PER_TILE)],
                    deg_out.at[c, pl.ds(r0, ROWS_PER_TILE)])


_sc_deg = functools.partial(
    pl.kernel,
    out_type=jax.ShapeDtypeStruct((NC, N_PAD, D_FEAT), jnp.float32),
    mesh=plsc.VectorSubcoreMesh(core_axis_name="c", subcore_axis_name="s"),
    scratch_types=(
        pltpu.VMEM((2 * GRP, CHUNK), jnp.int32),
        pltpu.VMEM((CHUNK, D_FEAT), jnp.float32),
        pltpu.VMEM_SHARED((N_PAD, D_FEAT), jnp.float32),
    ),
)(_sc_deg_body)


def _tc_body(h_ref, s_ref, d_ref, w_ref, b_ref, o_ref):
    hn = s_ref[0] + s_ref[1]
    dg = d_ref[0][:, :1] + d_ref[1][:, :1]
    hn = hn / jnp.maximum(dg, 1.0)
    w = w_ref[...]
    acc = lax.dot_general(h_ref[...], w[:, :D_FEAT], (((1,), (1,)), ((), ())),
                          preferred_element_type=jnp.float32)
    acc = acc + lax.dot_general(hn, w[:, D_FEAT:], (((1,), (1,)), ((), ())),
                                preferred_element_type=jnp.float32)
    o_ref[...] = acc + b_ref[...]


def _tc_linear(h, sums, deg, W, b2):
    blk = 1000
    grid = N_NODES // blk
    return pl.pallas_call(
        _tc_body,
        grid=(grid,),
        in_specs=[
            pl.BlockSpec((blk, D_FEAT), lambda i: (i, 0)),
            pl.BlockSpec((NC, blk, D_FEAT), lambda i: (0, i, 0)),
            pl.BlockSpec((NC, blk, D_FEAT), lambda i: (0, i, 0)),
            pl.BlockSpec((D_OUT, 2 * D_FEAT), lambda i: (0, 0)),
            pl.BlockSpec((1, D_OUT), lambda i: (0, 0)),
        ],
        out_specs=pl.BlockSpec((blk, D_OUT), lambda i: (i, 0)),
        out_shape=jax.ShapeDtypeStruct((N_NODES, D_OUT), jnp.float32),
    )(h, sums, deg, W, b2)


def kernel(h, edge_index, W, b):
    src = edge_index[0].astype(jnp.int32)
    dst = edge_index[1].astype(jnp.int32)
    pad = E_PAD - N_EDGES
    src = jnp.pad(src, (0, pad))
    dst = jnp.pad(dst, (0, pad), constant_values=N_PAD - 1)
    src4 = src.reshape(NW, NGRP, GRP, CHUNK)
    dst4 = dst.reshape(NW, NGRP, GRP, CHUNK)
    idx4 = jnp.concatenate([src4, dst4], axis=2)
    zeros = jnp.zeros((CHUNK, D_FEAT), jnp.float32)
    ones = jnp.ones((CHUNK, D_FEAT), jnp.float32)
    sums = _sc_feat(h, idx4, zeros)
    deg = _sc_deg(idx4, zeros, ones)
    return _tc_linear(h, sums, deg, W, b.reshape(1, D_OUT))


# speedup vs baseline: 3.0985x; 1.0656x over previous
import functools

import jax
import jax.numpy as jnp
from jax import lax
from jax.experimental import pallas as pl
from jax.experimental.pallas import tpu as pltpu
from jax.experimental.pallas import tpu_sc as plsc

N_NODES = 10000
N_EDGES = 320000
D_FEAT = 128
D_OUT = 128

NC = 2
NS = 16
NW = NC * NS
CHUNK = 64
CPT = 160
GRP = 8
NGRP = CPT // GRP
E_PAD = NW * CPT * CHUNK
N_PAD = 10112
ROWS_PER_TILE = N_PAD // NS


def _sc_feat_body(h_hbm, idx_hbm, zeros_hbm, sums_out, si_di, rb0, rb1,
                  acc_sh, sem0, sem1):
    c = lax.axis_index("c")
    s = lax.axis_index("s")
    wid = c * NS + s

    pltpu.sync_copy(zeros_hbm, rb0)
    r0 = s * ROWS_PER_TILE
    nblk = (ROWS_PER_TILE + CHUNK - 1) // CHUNK
    for k in range(nblk):
        n = min(CHUNK, ROWS_PER_TILE - CHUNK * k)
        pltpu.sync_copy(rb0.at[pl.ds(0, n)],
                        acc_sh.at[pl.ds(r0 + CHUNK * k, n)])

    plsc.subcore_barrier()

    bufs = (rb0, rb1)
    sems = (sem0, sem1)

    def _group(g, _):
        pltpu.sync_copy(idx_hbm.at[wid, g], si_di)
        cps = [pltpu.async_copy(h_hbm.at[si_di.at[0]], rb0, sem0), None]
        for j in range(GRP):
            cur = j % 2
            cps[cur].wait()
            if j + 1 < GRP:
                cps[1 - cur] = pltpu.async_copy(
                    h_hbm.at[si_di.at[j + 1]], bufs[1 - cur], sems[1 - cur])
            pltpu.sync_copy(bufs[cur], acc_sh.at[si_di.at[GRP + j]],
                            add=True)
        return 0

    lax.fori_loop(0, NGRP, _group, 0)

    plsc.subcore_barrier()

    pltpu.sync_copy(acc_sh.at[pl.ds(r0, ROWS_PER_TILE)],
                    sums_out.at[c, pl.ds(r0, ROWS_PER_TILE)])


_sc_feat = functools.partial(
    pl.kernel,
    out_type=jax.ShapeDtypeStruct((NC, N_PAD, D_FEAT), jnp.float32),
    mesh=plsc.VectorSubcoreMesh(core_axis_name="c", subcore_axis_name="s"),
    scratch_types=(
        pltpu.VMEM((2 * GRP, CHUNK), jnp.int32),
        pltpu.VMEM((CHUNK, D_FEAT), jnp.float32),
        pltpu.VMEM((CHUNK, D_FEAT), jnp.float32),
        pltpu.VMEM_SHARED((N_PAD, D_FEAT), jnp.float32),
        pltpu.SemaphoreType.DMA,
        pltpu.SemaphoreType.DMA,
    ),
)(_sc_feat_body)


def _sc_deg_body(idx_hbm, zeros_hbm, ones_hbm, deg_out, si_di, ob, deg_sh):
    c = lax.axis_index("c")
    s = lax.axis_index("s")
    wid = c * NS + s

    pltpu.sync_copy(zeros_hbm, ob)
    r0 = s * ROWS_PER_TILE
    nblk = (ROWS_PER_TILE + CHUNK - 1) // CHUNK
    for k in range(nblk):
        n = min(CHUNK, ROWS_PER_TILE - CHUNK * k)
        pltpu.sync_copy(ob.at[pl.ds(0, n)],
                        deg_sh.at[pl.ds(r0 + CHUNK * k, n)])
    pltpu.sync_copy(ones_hbm, ob)

    plsc.subcore_barrier()

    def _group(g, _):
        pltpu.sync_copy(idx_hbm.at[wid, g], si_di)
        for j in range(GRP):
            pltpu.sync_copy(ob, deg_sh.at[si_di.at[GRP + j]], add=True)
        return 0

    lax.fori_loop(0, NGRP, _group, 0)

    plsc.subcore_barrier()

    pltpu.sync_copy(deg_sh.at[pl.ds(r0, ROWS_PER_TILE)],
                    deg_out.at[c, pl.ds(r0, ROWS_PER_TILE)])


_sc_deg = functools.partial(
    pl.kernel,
    out_type=jax.ShapeDtypeStruct((NC, N_PAD, D_FEAT), jnp.float32),
    mesh=plsc.VectorSubcoreMesh(core_axis_name="c", subcore_axis_name="s"),
    scratch_types=(
        pltpu.VMEM((2 * GRP, CHUNK), jnp.int32),
        pltpu.VMEM((CHUNK, D_FEAT), jnp.float32),
        pltpu.VMEM_SHARED((N_PAD, D_FEAT), jnp.float32),
    ),
)(_sc_deg_body)


def _tc_body(h_ref, s_ref, d_ref, w_ref, b_ref, o_ref):
    hn = s_ref[0] + s_ref[1]
    dg = d_ref[0][:, :1] + d_ref[1][:, :1]
    hn = hn / jnp.maximum(dg, 1.0)
    w = w_ref[...]
    acc = lax.dot_general(h_ref[...], w[:, :D_FEAT], (((1,), (1,)), ((), ())),
                          preferred_element_type=jnp.float32)
    acc = acc + lax.dot_general(hn, w[:, D_FEAT:], (((1,), (1,)), ((), ())),
                                preferred_element_type=jnp.float32)
    o_ref[...] = acc + b_ref[...]


def _tc_linear(h, sums, deg, W, b2):
    blk = 1000
    grid = N_NODES // blk
    return pl.pallas_call(
        _tc_body,
        grid=(grid,),
        in_specs=[
            pl.BlockSpec((blk, D_FEAT), lambda i: (i, 0)),
            pl.BlockSpec((NC, blk, D_FEAT), lambda i: (0, i, 0)),
            pl.BlockSpec((NC, blk, D_FEAT), lambda i: (0, i, 0)),
            pl.BlockSpec((D_OUT, 2 * D_FEAT), lambda i: (0, 0)),
            pl.BlockSpec((1, D_OUT), lambda i: (0, 0)),
        ],
        out_specs=pl.BlockSpec((blk, D_OUT), lambda i: (i, 0)),
        out_shape=jax.ShapeDtypeStruct((N_NODES, D_OUT), jnp.float32),
    )(h, sums, deg, W, b2)


def kernel(h, edge_index, W, b):
    src = edge_index[0].astype(jnp.int32)
    dst = edge_index[1].astype(jnp.int32)
    pad = E_PAD - N_EDGES
    src = jnp.pad(src, (0, pad))
    dst = jnp.pad(dst, (0, pad), constant_values=N_PAD - 1)
    src4 = src.reshape(NW, NGRP, GRP, CHUNK)
    dst4 = dst.reshape(NW, NGRP, GRP, CHUNK)
    idx4 = jnp.concatenate([src4, dst4], axis=2)
    zeros = jnp.zeros((CHUNK, D_FEAT), jnp.float32)
    ones = jnp.ones((CHUNK, D_FEAT), jnp.float32)
    sums = _sc_feat(h, idx4, zeros)
    deg = _sc_deg(idx4, zeros, ones)
    return _tc_linear(h, sums, deg, W, b.reshape(1, D_OUT))
